# baseline (speedup 1.0000x reference)
import jax
import jax.numpy as jnp
from jax import lax
from jax.experimental import pallas as pl
from jax.experimental.pallas import tpu as pltpu
from jax.experimental.pallas import tpu_sc as plsc

_N = 2048
_NNZ = 32768
_LAYERS = 2

_NS = 16
_EPT = _NNZ // _NS
_NBLK = _EPT // 128
_BAND_ROWS = 512
_BAND_W = _BAND_ROWS * _N
_STRIDE_W = _BAND_W + 128
_SLICE_W = _BAND_W // _NS
_ZERO_W = 8192
_RING = 8


def _densify_body(rows_hbm, cols_hbm, va_hbm, vw_hbm, outa_hbm, outw_hbm,
                  rows_v, cols_v, va_v, vw_v, off2d, val2d, zero_v, sem,
                  band_sp):
    c = lax.axis_index("c")
    s = lax.axis_index("s")
    lanes = lax.iota(jnp.int32, 16)
    dump_lane = _BAND_W + s * 8 + lanes
    base = s * _EPT
    pltpu.sync_copy(rows_hbm.at[pl.ds(base, _EPT)], rows_v)
    pltpu.sync_copy(cols_hbm.at[pl.ds(base, _EPT)], cols_v)
    pltpu.sync_copy(va_hbm.at[pl.ds(base, _EPT)], va_v)
    pltpu.sync_copy(vw_hbm.at[pl.ds(base, _EPT)], vw_v)

    def zfill(i, _):
        zero_v[pl.ds(i * 16, 16)] = jnp.zeros((16,), jnp.float32)
        return 0
    lax.fori_loop(0, _ZERO_W // 16, zfill, 0)

    for vals_v, out_hbm in ((va_v, outa_hbm), (vw_v, outw_hbm)):
        for b_i in range(2):
            band = c * 2 + b_i

            for z in range(_SLICE_W // _ZERO_W):
                pltpu.sync_copy(
                    zero_v,
                    band_sp.at[pl.ds(s * _SLICE_W + z * _ZERO_W, _ZERO_W)])

            def scan(g, _):
                gg = g * 16
                r = rows_v[pl.ds(gg, 16)]
                cc = cols_v[pl.ds(gg, 16)]
                v = vals_v[pl.ds(gg, 16)]
                m = lax.shift_right_logical(r, 9) == band
                off = lax.shift_left(jnp.bitwise_and(r, 511), 11) + cc
                off = jnp.where(m, off, dump_lane)
                vv = jnp.where(m, v, jnp.float32(0.0))
                row = lax.shift_right_logical(g, 3)
                col = jnp.bitwise_and(g, 7) * 16
                off2d[row, pl.ds(col, 16)] = off
                val2d[row, pl.ds(col, 16)] = vv
                return 0
            lax.fori_loop(0, _EPT // 16, scan, 0)

            plsc.subcore_barrier()

            def round_body(q, _):
                @pl.when(s == q)
                def _mine():
                    descs = []
                    for j in range(_NBLK):
                        descs.append(pltpu.async_copy(
                            val2d.at[j], band_sp.at[off2d.at[j]], sem,
                            add=True))
                        if j >= _RING:
                            descs[j - _RING].wait()
                    for d in descs[_NBLK - _RING:]:
                        d.wait()
                plsc.subcore_barrier()
                return 0
            lax.fori_loop(0, _NS, round_body, 0)

            pltpu.sync_copy(
                band_sp.at[pl.ds(s * _SLICE_W, _SLICE_W)],
                out_hbm.at[pl.ds(band * _BAND_W + s * _SLICE_W, _SLICE_W)])

            plsc.subcore_barrier()


_densify = pl.kernel(
    _densify_body,
    out_type=[jax.ShapeDtypeStruct((_N * _N,), jnp.float32),
              jax.ShapeDtypeStruct((_N * _N,), jnp.float32)],
    mesh=plsc.VectorSubcoreMesh(core_axis_name="c", subcore_axis_name="s"),
    scratch_types=[
        pltpu.VMEM((_EPT,), jnp.int32),
        pltpu.VMEM((_EPT,), jnp.int32),
        pltpu.VMEM((_EPT,), jnp.float32),
        pltpu.VMEM((_EPT,), jnp.float32),
        pltpu.VMEM((_NBLK, 128), jnp.int32),
        pltpu.VMEM((_NBLK, 128), jnp.float32),
        pltpu.VMEM((_ZERO_W,), jnp.float32),
        pltpu.SemaphoreType.DMA,
        pltpu.VMEM_SHARED((_STRIDE_W,), jnp.float32),
    ],
)


_BM = 2048
_BN = 2048


def _mm_body(a_ref, b_ref, bias_ref, o_ref):
    o_ref[...] = jnp.dot(a_ref[...], b_ref[...],
                         preferred_element_type=jnp.float32) + bias_ref[...]


def _mm(a, b, bias_row):
    grid = (_N // _BM, _N // _BN)
    return pl.pallas_call(
        _mm_body,
        grid=grid,
        in_specs=[
            pl.BlockSpec((_BM, _N), lambda i, j: (i, 0)),
            pl.BlockSpec((_N, _BN), lambda i, j: (0, j)),
            pl.BlockSpec((1, _BN), lambda i, j: (0, j)),
        ],
        out_specs=pl.BlockSpec((_BM, _BN), lambda i, j: (i, j)),
        out_shape=jax.ShapeDtypeStruct((_N, _N), jnp.float32),
    )(a, b, bias_row)


def kernel(x, rows, cols, adj_vals, W_vals, bias):
    ad_flat, wd_flat = _densify(rows, cols, adj_vals, W_vals)
    a_d = ad_flat.reshape(_N, _N)
    w_d = wd_flat.reshape(_N, _N)
    zero_row = jnp.zeros((1, _N), jnp.float32)
    bias_row = bias.reshape(1, _N)
    m = _mm(a_d, w_d, zero_row)
    h = x
    for _ in range(_LAYERS):
        h = _mm(m, h, bias_row)
    return h


# speedup vs baseline: 1.0637x; 1.0637x over previous
import jax
import jax.numpy as jnp
from jax import lax
from jax.experimental import pallas as pl
from jax.experimental.pallas import tpu as pltpu
from jax.experimental.pallas import tpu_sc as plsc

_N = 2048
_NNZ = 32768
_LAYERS = 2

_NS = 16
_EPT = _NNZ // _NS
_NBLK = _EPT // 128
_BAND_ROWS = 512
_BAND_W = _BAND_ROWS * _N
_STRIDE_W = _BAND_W + 128
_SLICE_W = _BAND_W // _NS
_ZERO_W = 8192
_RING = 8


def _densify_body(rows_hbm, cols_hbm, va_hbm, vw_hbm, outa_hbm, outw_hbm,
                  rows_v, cols_v, va_v, vw_v, off2d, val2d, zero_v, sem,
                  band_sp):
    c = lax.axis_index("c")
    s = lax.axis_index("s")
    lanes = lax.iota(jnp.int32, 16)
    dump_lane = _BAND_W + s * 8 + lanes
    base = s * _EPT
    pltpu.sync_copy(rows_hbm.at[pl.ds(base, _EPT)], rows_v)
    pltpu.sync_copy(cols_hbm.at[pl.ds(base, _EPT)], cols_v)
    pltpu.sync_copy(va_hbm.at[pl.ds(base, _EPT)], va_v)
    pltpu.sync_copy(vw_hbm.at[pl.ds(base, _EPT)], vw_v)

    def zfill(i, _):
        zero_v[pl.ds(i * 16, 16)] = jnp.zeros((16,), jnp.float32)
        return 0
    lax.fori_loop(0, _ZERO_W // 16, zfill, 0)

    for vals_v, out_hbm in ((va_v, outa_hbm), (vw_v, outw_hbm)):
        for b_i in range(2):
            band = c * 2 + b_i

            for z in range(_SLICE_W // _ZERO_W):
                pltpu.sync_copy(
                    zero_v,
                    band_sp.at[pl.ds(s * _SLICE_W + z * _ZERO_W, _ZERO_W)])

            def scan(g, _):
                gg = g * 16
                r = rows_v[pl.ds(gg, 16)]
                cc = cols_v[pl.ds(gg, 16)]
                v = vals_v[pl.ds(gg, 16)]
                m = lax.shift_right_logical(r, 9) == band
                off = lax.shift_left(jnp.bitwise_and(r, 511), 11) + cc
                off = jnp.where(m, off, dump_lane)
                vv = jnp.where(m, v, jnp.float32(0.0))
                row = lax.shift_right_logical(g, 3)
                col = jnp.bitwise_and(g, 7) * 16
                off2d[row, pl.ds(col, 16)] = off
                val2d[row, pl.ds(col, 16)] = vv
                return 0
            lax.fori_loop(0, _EPT // 16, scan, 0)

            plsc.subcore_barrier()

            def round_body(q, _):
                @pl.when(s == q)
                def _mine():
                    descs = []
                    for j in range(_NBLK):
                        descs.append(pltpu.async_copy(
                            val2d.at[j], band_sp.at[off2d.at[j]], sem,
                            add=True))
                        if j >= _RING:
                            descs[j - _RING].wait()
                    for d in descs[_NBLK - _RING:]:
                        d.wait()
                plsc.subcore_barrier()
                return 0
            lax.fori_loop(0, _NS, round_body, 0)

            pltpu.sync_copy(
                band_sp.at[pl.ds(s * _SLICE_W, _SLICE_W)],
                out_hbm.at[pl.ds(band * _BAND_W + s * _SLICE_W, _SLICE_W)])

            plsc.subcore_barrier()


_densify = pl.kernel(
    _densify_body,
    out_type=[jax.ShapeDtypeStruct((_N * _N,), jnp.float32),
              jax.ShapeDtypeStruct((_N * _N,), jnp.float32)],
    mesh=plsc.VectorSubcoreMesh(core_axis_name="c", subcore_axis_name="s"),
    scratch_types=[
        pltpu.VMEM((_EPT,), jnp.int32),
        pltpu.VMEM((_EPT,), jnp.int32),
        pltpu.VMEM((_EPT,), jnp.float32),
        pltpu.VMEM((_EPT,), jnp.float32),
        pltpu.VMEM((_NBLK, 128), jnp.int32),
        pltpu.VMEM((_NBLK, 128), jnp.float32),
        pltpu.VMEM((_ZERO_W,), jnp.float32),
        pltpu.SemaphoreType.DMA,
        pltpu.VMEM_SHARED((_STRIDE_W,), jnp.float32),
    ],
)


_BM = 1024
_BN = 2048


def _mm_body(a_ref, b_ref, bias_ref, o_ref):
    o_ref[...] = jnp.dot(a_ref[...], b_ref[...],
                         preferred_element_type=jnp.float32) + bias_ref[...]


def _mm(a, b, bias_row):
    grid = (_N // _BM, _N // _BN)
    return pl.pallas_call(
        _mm_body,
        grid=grid,
        in_specs=[
            pl.BlockSpec((_BM, _N), lambda i, j: (i, 0)),
            pl.BlockSpec((_N, _BN), lambda i, j: (0, j)),
            pl.BlockSpec((1, _BN), lambda i, j: (0, j)),
        ],
        out_specs=pl.BlockSpec((_BM, _BN), lambda i, j: (i, j)),
        out_shape=jax.ShapeDtypeStruct((_N, _N), jnp.float32),
    )(a, b, bias_row)


def kernel(x, rows, cols, adj_vals, W_vals, bias):
    ad_flat, wd_flat = _densify(rows, cols, adj_vals, W_vals)
    a_d = ad_flat.reshape(_N, _N)
    w_d = wd_flat.reshape(_N, _N)
    zero_row = jnp.zeros((1, _N), jnp.float32)
    bias_row = bias.reshape(1, _N)
    m = _mm(a_d, w_d, zero_row)
    h = x
    for _ in range(_LAYERS):
        h = _mm(m, h, bias_row)
    return h


# speedup vs baseline: 1.0947x; 1.0291x over previous
import jax
import jax.numpy as jnp
from jax import lax
from jax.experimental import pallas as pl
from jax.experimental.pallas import tpu as pltpu
from jax.experimental.pallas import tpu_sc as plsc

_N = 2048
_NNZ = 32768
_LAYERS = 2

_NS = 16
_EPT = _NNZ // _NS
_NBLK = _EPT // 128
_BAND_ROWS = 512
_BAND_W = _BAND_ROWS * _N
_STRIDE_W = _BAND_W + 128
_SLICE_W = _BAND_W // _NS
_ZERO_W = 8192
_RING = 8


def _densify_body(rows_hbm, cols_hbm, va_hbm, vw_hbm, outa_hbm, outw_hbm,
                  rows_v, cols_v, va_v, vw_v, off2d, val2d, zero_v, sem,
                  band_sp):
    c = lax.axis_index("c")
    s = lax.axis_index("s")
    lanes = lax.iota(jnp.int32, 16)
    dump_lane = _BAND_W + s * 8 + lanes
    base = s * _EPT
    pltpu.sync_copy(rows_hbm.at[pl.ds(base, _EPT)], rows_v)
    pltpu.sync_copy(cols_hbm.at[pl.ds(base, _EPT)], cols_v)
    pltpu.sync_copy(va_hbm.at[pl.ds(base, _EPT)], va_v)
    pltpu.sync_copy(vw_hbm.at[pl.ds(base, _EPT)], vw_v)

    def zfill(i, _):
        zero_v[pl.ds(i * 16, 16)] = jnp.zeros((16,), jnp.float32)
        return 0
    lax.fori_loop(0, _ZERO_W // 16, zfill, 0)

    for vals_v, out_hbm in ((va_v, outa_hbm), (vw_v, outw_hbm)):
        for b_i in range(2):
            band = c * 2 + b_i

            for z in range(_SLICE_W // _ZERO_W):
                pltpu.sync_copy(
                    zero_v,
                    band_sp.at[pl.ds(s * _SLICE_W + z * _ZERO_W, _ZERO_W)])

            def scan(g, _):
                gg = g * 16
                r = rows_v[pl.ds(gg, 16)]
                cc = cols_v[pl.ds(gg, 16)]
                v = vals_v[pl.ds(gg, 16)]
                m = lax.shift_right_logical(r, 9) == band
                off = lax.shift_left(jnp.bitwise_and(r, 511), 11) + cc
                off = jnp.where(m, off, dump_lane)
                vv = jnp.where(m, v, jnp.float32(0.0))
                row = lax.shift_right_logical(g, 3)
                col = jnp.bitwise_and(g, 7) * 16
                off2d[row, pl.ds(col, 16)] = off
                val2d[row, pl.ds(col, 16)] = vv
                return 0
            lax.fori_loop(0, _EPT // 16, scan, 0)

            plsc.subcore_barrier()

            def round_body(q, _):
                @pl.when(s == q)
                def _mine():
                    descs = []
                    for j in range(_NBLK):
                        descs.append(pltpu.async_copy(
                            val2d.at[j], band_sp.at[off2d.at[j]], sem,
                            add=True))
                        if j >= _RING:
                            descs[j - _RING].wait()
                    for d in descs[_NBLK - _RING:]:
                        d.wait()
                plsc.subcore_barrier()
                return 0
            lax.fori_loop(0, _NS, round_body, 0)

            pltpu.sync_copy(
                band_sp.at[pl.ds(s * _SLICE_W, _SLICE_W)],
                out_hbm.at[pl.ds(band * _BAND_W + s * _SLICE_W, _SLICE_W)])

            plsc.subcore_barrier()


_densify = pl.kernel(
    _densify_body,
    out_type=[jax.ShapeDtypeStruct((_N * _N,), jnp.float32),
              jax.ShapeDtypeStruct((_N * _N,), jnp.float32)],
    mesh=plsc.VectorSubcoreMesh(core_axis_name="c", subcore_axis_name="s"),
    scratch_types=[
        pltpu.VMEM((_EPT,), jnp.int32),
        pltpu.VMEM((_EPT,), jnp.int32),
        pltpu.VMEM((_EPT,), jnp.float32),
        pltpu.VMEM((_EPT,), jnp.float32),
        pltpu.VMEM((_NBLK, 128), jnp.int32),
        pltpu.VMEM((_NBLK, 128), jnp.float32),
        pltpu.VMEM((_ZERO_W,), jnp.float32),
        pltpu.SemaphoreType.DMA,
        pltpu.VMEM_SHARED((_STRIDE_W,), jnp.float32),
    ],
)


_BM = 512
_BN = 2048


def _mm_body(a_ref, b_ref, bias_ref, o_ref):
    o_ref[...] = jnp.dot(a_ref[...], b_ref[...],
                         preferred_element_type=jnp.float32) + bias_ref[...]


def _mm(a, b, bias_row):
    grid = (_N // _BM, _N // _BN)
    return pl.pallas_call(
        _mm_body,
        grid=grid,
        in_specs=[
            pl.BlockSpec((_BM, _N), lambda i, j: (i, 0)),
            pl.BlockSpec((_N, _BN), lambda i, j: (0, j)),
            pl.BlockSpec((1, _BN), lambda i, j: (0, j)),
        ],
        out_specs=pl.BlockSpec((_BM, _BN), lambda i, j: (i, j)),
        out_shape=jax.ShapeDtypeStruct((_N, _N), jnp.float32),
    )(a, b, bias_row)


def kernel(x, rows, cols, adj_vals, W_vals, bias):
    ad_flat, wd_flat = _densify(rows, cols, adj_vals, W_vals)
    a_d = ad_flat.reshape(_N, _N)
    w_d = wd_flat.reshape(_N, _N)
    zero_row = jnp.zeros((1, _N), jnp.float32)
    bias_row = bias.reshape(1, _N)
    m = _mm(a_d, w_d, zero_row)
    h = x
    for _ in range(_LAYERS):
        h = _mm(m, h, bias_row)
    return h


# speedup vs baseline: 1.0948x; 1.0001x over previous
import jax
import jax.numpy as jnp
from jax import lax
from jax.experimental import pallas as pl
from jax.experimental.pallas import tpu as pltpu
from jax.experimental.pallas import tpu_sc as plsc

_N = 2048
_NNZ = 32768
_LAYERS = 2

_NS = 16
_EPT = _NNZ // _NS
_NBLK = _EPT // 128
_BAND_ROWS = 512
_BAND_W = _BAND_ROWS * _N
_STRIDE_W = _BAND_W + 128
_SLICE_W = _BAND_W // _NS
_ZERO_W = 8192
_RING = 8


def _densify_body(rows_hbm, cols_hbm, va_hbm, vw_hbm, outa_hbm, outw_hbm,
                  rows_v, cols_v, va_v, vw_v, off2d, val2d, zero_v, sem,
                  band_sp):
    c = lax.axis_index("c")
    s = lax.axis_index("s")
    lanes = lax.iota(jnp.int32, 16)
    dump_lane = _BAND_W + s * 8 + lanes
    base = s * _EPT
    pltpu.sync_copy(rows_hbm.at[pl.ds(base, _EPT)], rows_v)
    pltpu.sync_copy(cols_hbm.at[pl.ds(base, _EPT)], cols_v)
    pltpu.sync_copy(va_hbm.at[pl.ds(base, _EPT)], va_v)
    pltpu.sync_copy(vw_hbm.at[pl.ds(base, _EPT)], vw_v)

    def zfill(i, _):
        zero_v[pl.ds(i * 16, 16)] = jnp.zeros((16,), jnp.float32)
        return 0
    lax.fori_loop(0, _ZERO_W // 16, zfill, 0)

    for vals_v, out_hbm in ((va_v, outa_hbm), (vw_v, outw_hbm)):
        for b_i in range(2):
            band = c * 2 + b_i

            for z in range(_SLICE_W // _ZERO_W):
                pltpu.sync_copy(
                    zero_v,
                    band_sp.at[pl.ds(s * _SLICE_W + z * _ZERO_W, _ZERO_W)])

            def scan(g, _):
                gg = g * 16
                r = rows_v[pl.ds(gg, 16)]
                cc = cols_v[pl.ds(gg, 16)]
                v = vals_v[pl.ds(gg, 16)]
                m = lax.shift_right_logical(r, 9) == band
                off = lax.shift_left(jnp.bitwise_and(r, 511), 11) + cc
                off = jnp.where(m, off, dump_lane)
                vv = jnp.where(m, v, jnp.float32(0.0))
                row = lax.shift_right_logical(g, 3)
                col = jnp.bitwise_and(g, 7) * 16
                off2d[row, pl.ds(col, 16)] = off
                val2d[row, pl.ds(col, 16)] = vv
                return 0
            lax.fori_loop(0, _EPT // 16, scan, 0)

            plsc.subcore_barrier()

            def round_body(q, _):
                @pl.when(s == q)
                def _mine():
                    descs = []
                    for j in range(_NBLK):
                        descs.append(pltpu.async_copy(
                            val2d.at[j], band_sp.at[off2d.at[j]], sem,
                            add=True))
                        if j >= _RING:
                            descs[j - _RING].wait()
                    for d in descs[_NBLK - _RING:]:
                        d.wait()
                plsc.subcore_barrier()
                return 0
            lax.fori_loop(0, _NS, round_body, 0)

            pltpu.sync_copy(
                band_sp.at[pl.ds(s * _SLICE_W, _SLICE_W)],
                out_hbm.at[pl.ds(band * _BAND_W + s * _SLICE_W, _SLICE_W)])

            plsc.subcore_barrier()


_densify = pl.kernel(
    _densify_body,
    out_type=[jax.ShapeDtypeStruct((_N * _N,), jnp.float32),
              jax.ShapeDtypeStruct((_N * _N,), jnp.float32)],
    mesh=plsc.VectorSubcoreMesh(core_axis_name="c", subcore_axis_name="s"),
    scratch_types=[
        pltpu.VMEM((_EPT,), jnp.int32),
        pltpu.VMEM((_EPT,), jnp.int32),
        pltpu.VMEM((_EPT,), jnp.float32),
        pltpu.VMEM((_EPT,), jnp.float32),
        pltpu.VMEM((_NBLK, 128), jnp.int32),
        pltpu.VMEM((_NBLK, 128), jnp.float32),
        pltpu.VMEM((_ZERO_W,), jnp.float32),
        pltpu.SemaphoreType.DMA,
        pltpu.VMEM_SHARED((_STRIDE_W,), jnp.float32),
    ],
)


_BM = 256
_BN = 2048


def _mm_body(a_ref, b_ref, bias_ref, o_ref):
    o_ref[...] = jnp.dot(a_ref[...], b_ref[...],
                         preferred_element_type=jnp.float32) + bias_ref[...]


def _mm(a, b, bias_row):
    grid = (_N // _BM, _N // _BN)
    return pl.pallas_call(
        _mm_body,
        grid=grid,
        in_specs=[
            pl.BlockSpec((_BM, _N), lambda i, j: (i, 0)),
            pl.BlockSpec((_N, _BN), lambda i, j: (0, j)),
            pl.BlockSpec((1, _BN), lambda i, j: (0, j)),
        ],
        out_specs=pl.BlockSpec((_BM, _BN), lambda i, j: (i, j)),
        out_shape=jax.ShapeDtypeStruct((_N, _N), jnp.float32),
    )(a, b, bias_row)


def kernel(x, rows, cols, adj_vals, W_vals, bias):
    ad_flat, wd_flat = _densify(rows, cols, adj_vals, W_vals)
    a_d = ad_flat.reshape(_N, _N)
    w_d = wd_flat.reshape(_N, _N)
    zero_row = jnp.zeros((1, _N), jnp.float32)
    bias_row = bias.reshape(1, _N)
    m = _mm(a_d, w_d, zero_row)
    h = x
    for _ in range(_LAYERS):
        h = _mm(m, h, bias_row)
    return h


# speedup vs baseline: 1.1034x; 1.0078x over previous
import jax
import jax.numpy as jnp
from jax import lax
from jax.experimental import pallas as pl
from jax.experimental.pallas import tpu as pltpu
from jax.experimental.pallas import tpu_sc as plsc

_N = 2048
_NNZ = 32768
_LAYERS = 2

_NS = 16
_EPT = _NNZ // _NS
_NBLK = _EPT // 128
_BAND_ROWS = 512
_BAND_W = _BAND_ROWS * _N
_STRIDE_W = _BAND_W + 128
_SLICE_W = _BAND_W // _NS
_ZERO_W = 8192
_RING = 8


def _densify_body(rows_hbm, cols_hbm, va_hbm, vw_hbm, outa_hbm, outw_hbm,
                  rows_v, cols_v, va_v, vw_v, off2d, val2d, zero_v, sem,
                  band_sp):
    c = lax.axis_index("c")
    s = lax.axis_index("s")
    lanes = lax.iota(jnp.int32, 16)
    dump_lane = _BAND_W + s * 8 + lanes
    base = s * _EPT
    pltpu.sync_copy(rows_hbm.at[pl.ds(base, _EPT)], rows_v)
    pltpu.sync_copy(cols_hbm.at[pl.ds(base, _EPT)], cols_v)
    pltpu.sync_copy(va_hbm.at[pl.ds(base, _EPT)], va_v)
    pltpu.sync_copy(vw_hbm.at[pl.ds(base, _EPT)], vw_v)

    def zfill(i, _):
        zero_v[pl.ds(i * 16, 16)] = jnp.zeros((16,), jnp.float32)
        return 0
    lax.fori_loop(0, _ZERO_W // 16, zfill, 0)

    for vals_v, out_hbm in ((va_v, outa_hbm), (vw_v, outw_hbm)):
        for b_i in range(2):
            band = c * 2 + b_i

            zdescs = [
                pltpu.async_copy(
                    zero_v,
                    band_sp.at[pl.ds(s * _SLICE_W + z * _ZERO_W, _ZERO_W)],
                    sem)
                for z in range(_SLICE_W // _ZERO_W)]
            for d in zdescs:
                d.wait()

            def scan(g, _):
                gg = g * 16
                r = rows_v[pl.ds(gg, 16)]
                cc = cols_v[pl.ds(gg, 16)]
                v = vals_v[pl.ds(gg, 16)]
                m = lax.shift_right_logical(r, 9) == band
                off = lax.shift_left(jnp.bitwise_and(r, 511), 11) + cc
                off = jnp.where(m, off, dump_lane)
                vv = jnp.where(m, v, jnp.float32(0.0))
                row = lax.shift_right_logical(g, 3)
                col = jnp.bitwise_and(g, 7) * 16
                off2d[row, pl.ds(col, 16)] = off
                val2d[row, pl.ds(col, 16)] = vv
                return 0
            lax.fori_loop(0, _EPT // 16, scan, 0)

            plsc.subcore_barrier()

            def round_body(q, _):
                @pl.when(s == q)
                def _mine():
                    descs = []
                    for j in range(_NBLK):
                        descs.append(pltpu.async_copy(
                            val2d.at[j], band_sp.at[off2d.at[j]], sem,
                            add=True))
                        if j >= _RING:
                            descs[j - _RING].wait()
                    for d in descs[_NBLK - _RING:]:
                        d.wait()
                plsc.subcore_barrier()
                return 0
            lax.fori_loop(0, _NS, round_body, 0)

            pltpu.sync_copy(
                band_sp.at[pl.ds(s * _SLICE_W, _SLICE_W)],
                out_hbm.at[pl.ds(band * _BAND_W + s * _SLICE_W, _SLICE_W)])

            plsc.subcore_barrier()


_densify = pl.kernel(
    _densify_body,
    out_type=[jax.ShapeDtypeStruct((_N * _N,), jnp.float32),
              jax.ShapeDtypeStruct((_N * _N,), jnp.float32)],
    mesh=plsc.VectorSubcoreMesh(core_axis_name="c", subcore_axis_name="s"),
    scratch_types=[
        pltpu.VMEM((_EPT,), jnp.int32),
        pltpu.VMEM((_EPT,), jnp.int32),
        pltpu.VMEM((_EPT,), jnp.float32),
        pltpu.VMEM((_EPT,), jnp.float32),
        pltpu.VMEM((_NBLK, 128), jnp.int32),
        pltpu.VMEM((_NBLK, 128), jnp.float32),
        pltpu.VMEM((_ZERO_W,), jnp.float32),
        pltpu.SemaphoreType.DMA,
        pltpu.VMEM_SHARED((_STRIDE_W,), jnp.float32),
    ],
)


_BM = 256
_BN = 2048


def _mm_body(a_ref, b_ref, bias_ref, o_ref):
    o_ref[...] = jnp.dot(a_ref[...], b_ref[...],
                         preferred_element_type=jnp.float32) + bias_ref[...]


def _mm(a, b, bias_row):
    grid = (_N // _BM, _N // _BN)
    return pl.pallas_call(
        _mm_body,
        grid=grid,
        in_specs=[
            pl.BlockSpec((_BM, _N), lambda i, j: (i, 0)),
            pl.BlockSpec((_N, _BN), lambda i, j: (0, j)),
            pl.BlockSpec((1, _BN), lambda i, j: (0, j)),
        ],
        out_specs=pl.BlockSpec((_BM, _BN), lambda i, j: (i, j)),
        out_shape=jax.ShapeDtypeStruct((_N, _N), jnp.float32),
    )(a, b, bias_row)


def kernel(x, rows, cols, adj_vals, W_vals, bias):
    ad_flat, wd_flat = _densify(rows, cols, adj_vals, W_vals)
    a_d = ad_flat.reshape(_N, _N)
    w_d = wd_flat.reshape(_N, _N)
    zero_row = jnp.zeros((1, _N), jnp.float32)
    bias_row = bias.reshape(1, _N)
    m = _mm(a_d, w_d, zero_row)
    h = x
    for _ in range(_LAYERS):
        h = _mm(m, h, bias_row)
    return h
